# baseline (speedup 1.0000x reference)
import dataclasses

import jax
import jax.numpy as jnp
from jax import lax
from jax.experimental import pallas as pl
from jax.experimental.pallas import tpu as pltpu
from jax.experimental.pallas import tpu_sc as plsc

N_FACTORS = 32
N_CLASSES = 5
OUT_PAD = 8
BATCH = 16384
NUM_WORKERS = 32
ROWS_PER_WORKER = BATCH // NUM_WORKERS
CHUNK = 128
NUM_CHUNKS = ROWS_PER_WORKER // CHUNK
LANES = 16
CHUNKS16_PER_GROUP = 4
GROUP_ROWS = LANES * CHUNKS16_PER_GROUP
GROUPS_PER_CHUNK = CHUNK // GROUP_ROWS
PACK = 128 // N_FACTORS
WIDE = 128


def _sc_body(user_ref, item_ref, ut_ref, it_ref, w_ref, b_ref, out_ref,
             uidx, vidx, ubnd, vbnd, ucol, vcol,
             u_bufs, v_bufs, w_vmem, b_vmem, wb_vmem, out_buf, sems):
    core = lax.axis_index("core")
    subcore = lax.axis_index("subcore")
    wid = subcore * 2 + core
    base = wid * ROWS_PER_WORKER

    pltpu.sync_copy(user_ref.at[pl.ds(base, ROWS_PER_WORKER)], uidx)
    pltpu.sync_copy(item_ref.at[pl.ds(base, ROWS_PER_WORKER)], vidx)
    pltpu.sync_copy(w_ref, w_vmem)
    pltpu.sync_copy(b_ref, b_vmem)

    @pl.loop(0, ROWS_PER_WORKER // LANES)
    def _(i):
        sl = pl.ds(i * LANES, LANES)
        u = uidx[sl]
        v = vidx[sl]
        ubnd[sl] = jax.lax.shift_right_logical(u, 2)
        vbnd[sl] = jax.lax.shift_right_logical(v, 2)
        ucol[sl] = jax.lax.shift_left(u & 3, 5)
        vcol[sl] = jax.lax.shift_left(v & 3, 5)

    def fire(j):
        slot = j % 2
        sl = pl.ds(j * CHUNK, CHUNK)
        return (
            pltpu.async_copy(ut_ref.at[ubnd.at[sl]], u_bufs.at[slot], sems[slot]),
            pltpu.async_copy(it_ref.at[vbnd.at[sl]], v_bufs.at[slot], sems[2 + slot]),
        )

    inflight = {0: fire(0), 1: fire(1)}

    @pl.loop(0, N_FACTORS * N_CLASSES)
    def _(k):
        kv = jnp.full((LANES,), 0, jnp.int32) + k
        wb_vmem[pl.ds(k * LANES, LANES)] = plsc.load_gather(w_vmem, [kv])

    bias = [plsc.load_gather(b_vmem, [jnp.full((LANES,), 8 + c, jnp.int32)])
            for c in range(N_CLASSES)]

    iota16 = lax.iota(jnp.int32, LANES)

    for j in range(NUM_CHUNKS):
        slot = j % 2
        for cp in inflight.pop(j):
            cp.wait()
        u_rows = u_bufs.at[slot]
        v_rows = v_bufs.at[slot]
        for g in range(GROUPS_PER_CHUNK):
            gbase = j * CHUNK + g * GROUP_ROWS
            rows = [iota16 + (g * GROUP_ROWS + r * LANES)
                    for r in range(CHUNKS16_PER_GROUP)]
            uc = [ucol[pl.ds(gbase + r * LANES, LANES)]
                  for r in range(CHUNKS16_PER_GROUP)]
            vc = [vcol[pl.ds(gbase + r * LANES, LANES)]
                  for r in range(CHUNKS16_PER_GROUP)]
            accs = [[bias[c] for c in range(N_CLASSES)]
                    for _ in range(CHUNKS16_PER_GROUP)]
            for f in range(N_FACTORS):
                wv = [wb_vmem[pl.ds((f * N_CLASSES + c) * LANES, LANES)]
                      for c in range(N_CLASSES)]
                for r in range(CHUNKS16_PER_GROUP):
                    uf = plsc.load_gather(u_rows, [rows[r], uc[r] + f])
                    vf = plsc.load_gather(v_rows, [rows[r], vc[r] + f])
                    feat = uf * vf
                    for c in range(N_CLASSES):
                        accs[r][c] = accs[r][c] + feat * wv[c]
            for r in range(CHUNKS16_PER_GROUP):
                for c in range(N_CLASSES):
                    out_buf[c, pl.ds(gbase + r * LANES, LANES)] = accs[r][c]
        if j + 2 < NUM_CHUNKS:
            inflight[j + 2] = fire(j + 2)

    pltpu.sync_copy(out_buf, out_ref.at[:, pl.ds(base, ROWS_PER_WORKER)])


def kernel(user, item, user_table, item_table, W, b):
    mesh = plsc.VectorSubcoreMesh(core_axis_name="core",
                                  subcore_axis_name="subcore")
    cp = pltpu.CompilerParams(use_tc_tiling_on_sc=True)
    if "needs_layout_passes" in pltpu.CompilerParams.__dataclass_fields__:
        cp = dataclasses.replace(cp, needs_layout_passes=False)
    k = pl.kernel(
        _sc_body,
        out_type=jax.ShapeDtypeStruct((OUT_PAD, BATCH), jnp.float32),
        mesh=mesh,
        compiler_params=cp,
        scratch_types=[
            pltpu.VMEM((ROWS_PER_WORKER,), jnp.int32),
            pltpu.VMEM((ROWS_PER_WORKER,), jnp.int32),
            pltpu.VMEM((ROWS_PER_WORKER,), jnp.int32),
            pltpu.VMEM((ROWS_PER_WORKER,), jnp.int32),
            pltpu.VMEM((ROWS_PER_WORKER,), jnp.int32),
            pltpu.VMEM((ROWS_PER_WORKER,), jnp.int32),
            pltpu.VMEM((2, CHUNK, WIDE), jnp.float32),
            pltpu.VMEM((2, CHUNK, WIDE), jnp.float32),
            pltpu.VMEM((N_FACTORS * N_CLASSES,), jnp.float32),
            pltpu.VMEM((LANES,), jnp.float32),
            pltpu.VMEM((N_FACTORS * N_CLASSES * LANES,), jnp.float32),
            pltpu.VMEM((OUT_PAD, ROWS_PER_WORKER), jnp.float32),
            [pltpu.SemaphoreType.DMA] * 4,
        ],
    )
    ut_b, it_b = jax.lax.optimization_barrier((user_table.T, item_table.T))
    ut = jax.lax.transpose(ut_b, (1, 0)).reshape(-1, PACK * N_FACTORS)
    it = jax.lax.transpose(it_b, (1, 0)).reshape(-1, PACK * N_FACTORS)
    b_pad = jnp.zeros((LANES,), jnp.float32).at[8:8 + N_CLASSES].set(b)
    out = k(user.astype(jnp.int32), item.astype(jnp.int32),
            ut, it, W.reshape(-1), b_pad)
    return out[:N_CLASSES].T
